# baseline (speedup 1.0000x reference)
import functools

import jax
import jax.numpy as jnp
from jax import lax
from jax.experimental import pallas as pl
from jax.experimental.pallas import tpu as pltpu
from jax.experimental.pallas import tpu_sc as plsc

VOCAB = 37
EMBED = 16
BATCH = 16384
SEQ = 200
N = BATCH * SEQ

NUM_CORES = 2
NUM_SUBCORES = 16
NW = NUM_CORES * NUM_SUBCORES
PER_W = N // NW
CHUNK = 2048
STEPS = PER_W // CHUNK
NBUF = 3

_mesh = plsc.VectorSubcoreMesh(core_axis_name="c", subcore_axis_name="s")


@functools.partial(
    pl.kernel,
    mesh=_mesh,
    out_type=jax.ShapeDtypeStruct((N, EMBED), jnp.float32),
    scratch_types=[
        pltpu.VMEM((NBUF, CHUNK), jnp.int32),
        pltpu.VMEM((NBUF, CHUNK, EMBED), jnp.float32),
        [pltpu.SemaphoreType.DMA] * NBUF,
        [pltpu.SemaphoreType.DMA] * NBUF,
    ],
    compiler_params=pltpu.CompilerParams(use_tc_tiling_on_sc=False),
)
def _embed_lookup(ids, table_hbm, out, idx_v, rows_v, gsems, osems):
    wid = lax.axis_index("s") * NUM_CORES + lax.axis_index("c")
    base = wid * PER_W

    def start_gather(step, b):
        off = base + step * CHUNK
        pltpu.sync_copy(ids.at[pl.ds(off, CHUNK)], idx_v.at[b])
        pltpu.async_copy(table_hbm.at[idx_v.at[b]], rows_v.at[b], gsems[b])

    def process(step, b, refill):
        off = base + step * CHUNK
        pltpu.make_async_copy(table_hbm.at[idx_v.at[b]], rows_v.at[b],
                              gsems[b]).wait()
        pltpu.async_copy(rows_v.at[b], out.at[pl.ds(off, CHUNK)], osems[b])
        if refill:
            pltpu.make_async_copy(rows_v.at[b], out.at[pl.ds(off, CHUNK)],
                                  osems[b]).wait()
            start_gather(step + NBUF, b)

    for b in range(NBUF):
        start_gather(b, b)

    num_groups = (STEPS - NBUF) // NBUF

    def outer(i, carry):
        for b in range(NBUF):
            process(i * NBUF + b, b, refill=True)
        return carry

    lax.fori_loop(0, num_groups, outer, 0)

    for step in range(num_groups * NBUF, STEPS):
        process(step, step % NBUF, refill=step + NBUF < STEPS)

    for b in range(NBUF):
        pltpu.make_async_copy(rows_v.at[b], out.at[pl.ds(0, CHUNK)],
                              osems[b]).wait()


def kernel(char_ids, embed_weight):
    ids = char_ids.reshape(N).astype(jnp.int32)
    out = _embed_lookup(ids, embed_weight.astype(jnp.float32))
    return out.reshape(BATCH, SEQ, EMBED)


# speedup vs baseline: 1.9423x; 1.9423x over previous
import functools

import jax
import jax.numpy as jnp
from jax import lax
from jax.experimental import pallas as pl
from jax.experimental.pallas import tpu as pltpu
from jax.experimental.pallas import tpu_sc as plsc

VOCAB = 37
EMBED = 16
BATCH = 16384
SEQ = 200
N = BATCH * SEQ

NUM_CORES = 2
NUM_SUBCORES = 16
NW = NUM_CORES * NUM_SUBCORES
PER_W = N // NW
CHUNK = 2048
STEPS = PER_W // CHUNK
NBUF = 2
GROUPS = CHUNK // 16

_mesh = plsc.VectorSubcoreMesh(core_axis_name="c", subcore_axis_name="s")


@functools.partial(
    pl.kernel,
    mesh=_mesh,
    out_type=jax.ShapeDtypeStruct((N * EMBED,), jnp.float32),
    scratch_types=[
        pltpu.VMEM((VOCAB * EMBED,), jnp.float32),
        pltpu.VMEM((NBUF, CHUNK), jnp.int32),
        pltpu.VMEM((NBUF, CHUNK * EMBED), jnp.float32),
        [pltpu.SemaphoreType.DMA] * NBUF,
        [pltpu.SemaphoreType.DMA] * NBUF,
        pltpu.SemaphoreType.DMA,
    ],
    compiler_params=pltpu.CompilerParams(use_tc_tiling_on_sc=False,
                                         needs_layout_passes=False),
)
def _embed_lookup(ids, table_hbm, out, tbl_v, idx_v, rows_v, isems, osems,
                  tsem):
    wid = lax.axis_index("s") * NUM_CORES + lax.axis_index("c")
    base = wid * PER_W

    pltpu.async_copy(table_hbm, tbl_v, tsem)
    for b in range(NBUF):
        pltpu.async_copy(ids.at[pl.ds(base + b * CHUNK, CHUNK)], idx_v.at[b],
                         isems[b])
    pltpu.make_async_copy(table_hbm, tbl_v, tsem).wait()

    iota16 = lax.iota(jnp.int32, 16)
    sv_const = iota16 * EMBED

    def compute_chunk(b):
        def group(g, carry):
            ids_v = idx_v.at[b][pl.ds(g * 16, 16)]
            gbase = ids_v * EMBED
            sbase = sv_const + g * (16 * EMBED)
            for d in range(EMBED):
                v = plsc.load_gather(tbl_v, [gbase + d])
                plsc.store_scatter(rows_v.at[b], [sbase + d], v)
            return carry
        lax.fori_loop(0, GROUPS, group, 0)

    def process(step, b, refill):
        off = base + step * CHUNK
        pltpu.make_async_copy(ids.at[pl.ds(off, CHUNK)], idx_v.at[b],
                              isems[b]).wait()
        @pl.when(step >= NBUF)
        def _():
            pltpu.make_async_copy(
                rows_v.at[b], out.at[pl.ds(0, CHUNK * EMBED)], osems[b]).wait()
        compute_chunk(b)
        pltpu.async_copy(rows_v.at[b],
                         out.at[pl.ds(off * EMBED, CHUNK * EMBED)], osems[b])
        if refill:
            pltpu.async_copy(
                ids.at[pl.ds(base + (step + NBUF) * CHUNK, CHUNK)],
                idx_v.at[b], isems[b])

    num_groups = (STEPS - NBUF) // NBUF

    def outer(i, carry):
        for b in range(NBUF):
            process(i * NBUF + b, b, refill=True)
        return carry

    lax.fori_loop(0, num_groups, outer, 0)

    for step in range(num_groups * NBUF, STEPS):
        process(step, step % NBUF, refill=step + NBUF < STEPS)

    for b in range(NBUF):
        pltpu.make_async_copy(rows_v.at[b], out.at[pl.ds(0, CHUNK * EMBED)],
                              osems[b]).wait()


def kernel(char_ids, embed_weight):
    ids = char_ids.reshape(N).astype(jnp.int32)
    out = _embed_lookup(ids, embed_weight.reshape(VOCAB * EMBED))
    return out.reshape(BATCH, SEQ, EMBED)


# speedup vs baseline: 1.9456x; 1.0017x over previous
import functools

import jax
import jax.numpy as jnp
from jax import lax
from jax.experimental import pallas as pl
from jax.experimental.pallas import tpu as pltpu
from jax.experimental.pallas import tpu_sc as plsc

VOCAB = 37
EMBED = 16
BATCH = 16384
SEQ = 200
N = BATCH * SEQ

NUM_CORES = 2
NUM_SUBCORES = 16
NW = NUM_CORES * NUM_SUBCORES
ROWS_W = BATCH // NW
CB = 8
CHUNK = CB * SEQ
STEPS = ROWS_W // CB
NBUF = 2
GROUPS = CHUNK // 16

_mesh = plsc.VectorSubcoreMesh(core_axis_name="c", subcore_axis_name="s")


@functools.partial(
    pl.kernel,
    mesh=_mesh,
    out_type=jax.ShapeDtypeStruct((BATCH, SEQ, EMBED), jnp.float32),
    scratch_types=[
        pltpu.VMEM((VOCAB * EMBED,), jnp.float32),
        pltpu.VMEM((NBUF, CHUNK), jnp.int32),
        pltpu.VMEM((NBUF, CB, SEQ, EMBED), jnp.float32),
        [pltpu.SemaphoreType.DMA] * NBUF,
        [pltpu.SemaphoreType.DMA] * NBUF,
        pltpu.SemaphoreType.DMA,
    ],
    compiler_params=pltpu.CompilerParams(use_tc_tiling_on_sc=False,
                                         needs_layout_passes=False),
)
def _embed_lookup(ids, table_hbm, out, tbl_v, idx_v, rows_v, isems, osems,
                  tsem):
    wid = lax.axis_index("s") * NUM_CORES + lax.axis_index("c")
    row_base = wid * ROWS_W
    id_base = row_base * SEQ

    pltpu.async_copy(table_hbm, tbl_v, tsem)
    for b in range(NBUF):
        pltpu.async_copy(ids.at[pl.ds(id_base + b * CHUNK, CHUNK)],
                         idx_v.at[b], isems[b])
    pltpu.make_async_copy(table_hbm, tbl_v, tsem).wait()

    iota16 = lax.iota(jnp.int32, 16)

    def compute_chunk(b):
        def group(g, carry):
            f = g * 16 + iota16
            bb = f // SEQ
            s = f - bb * SEQ
            ids_v = idx_v.at[b][pl.ds(g * 16, 16)]
            gbase = ids_v * EMBED
            for d in range(EMBED):
                v = plsc.load_gather(tbl_v, [gbase + d])
                plsc.store_scatter(rows_v.at[b],
                                   [bb, s, jnp.full((16,), d, jnp.int32)], v)
            return carry
        lax.fori_loop(0, GROUPS, group, 0)

    def process(step, b, refill):
        pltpu.make_async_copy(ids.at[pl.ds(0, CHUNK)], idx_v.at[b],
                              isems[b]).wait()
        @pl.when(step >= NBUF)
        def _():
            pltpu.make_async_copy(rows_v.at[b],
                                  out.at[pl.ds(0, CB)], osems[b]).wait()
        compute_chunk(b)
        pltpu.async_copy(rows_v.at[b],
                         out.at[pl.ds(row_base + step * CB, CB)], osems[b])
        if refill:
            pltpu.async_copy(
                ids.at[pl.ds(id_base + (step + NBUF) * CHUNK, CHUNK)],
                idx_v.at[b], isems[b])

    num_groups = (STEPS - NBUF) // NBUF

    def outer(i, carry):
        for b in range(NBUF):
            process(i * NBUF + b, b, refill=True)
        return carry

    lax.fori_loop(0, num_groups, outer, 0)

    for step in range(num_groups * NBUF, STEPS):
        process(step, step % NBUF, refill=step + NBUF < STEPS)

    for b in range(NBUF):
        pltpu.make_async_copy(rows_v.at[b], out.at[pl.ds(0, CB)],
                              osems[b]).wait()


def kernel(char_ids, embed_weight):
    ids = char_ids.reshape(N).astype(jnp.int32)
    return _embed_lookup(ids, embed_weight.reshape(VOCAB * EMBED))


# speedup vs baseline: 3.4602x; 1.7785x over previous
import functools

import jax
import jax.numpy as jnp
from jax import lax
from jax.experimental import pallas as pl
from jax.experimental.pallas import tpu as pltpu
from jax.experimental.pallas import tpu_sc as plsc

VOCAB = 37
EMBED = 16
BATCH = 16384
SEQ = 200
N = BATCH * SEQ

NUM_CORES = 2
NUM_SUBCORES = 16
NW = NUM_CORES * NUM_SUBCORES
ROWS_W = BATCH // NW
CB = 8
CHUNK = CB * SEQ
STEPS = ROWS_W // CB
NBUF = 2
GROUPS = CHUNK // 16

_mesh = plsc.VectorSubcoreMesh(core_axis_name="c", subcore_axis_name="s")


@functools.partial(
    pl.kernel,
    mesh=_mesh,
    out_type=jax.ShapeDtypeStruct((BATCH, EMBED, SEQ), jnp.float32),
    scratch_types=[
        pltpu.VMEM((VOCAB * EMBED,), jnp.float32),
        pltpu.VMEM((NBUF, CHUNK), jnp.int32),
        pltpu.VMEM((NBUF, CB, EMBED, SEQ), jnp.float32),
        [pltpu.SemaphoreType.DMA] * NBUF,
        [pltpu.SemaphoreType.DMA] * NBUF,
        pltpu.SemaphoreType.DMA,
    ],
    compiler_params=pltpu.CompilerParams(use_tc_tiling_on_sc=False,
                                         needs_layout_passes=False),
)
def _embed_lookup(ids, table_hbm, out, tbl_v, idx_v, rows_v, isems, osems,
                  tsem):
    wid = lax.axis_index("s") * NUM_CORES + lax.axis_index("c")
    row_base = wid * ROWS_W
    id_base = row_base * SEQ

    pltpu.async_copy(table_hbm, tbl_v, tsem)
    for b in range(NBUF):
        pltpu.async_copy(ids.at[pl.ds(id_base + b * CHUNK, CHUNK)],
                         idx_v.at[b], isems[b])
    pltpu.make_async_copy(table_hbm, tbl_v, tsem).wait()

    iota16 = lax.iota(jnp.int32, 16)

    def compute_chunk(b):
        def group(g, carry):
            f = g * 16 + iota16
            bb = f // SEQ
            s = f - bb * SEQ
            ids_v = idx_v.at[b][pl.ds(g * 16, 16)]
            gbase = ids_v * EMBED
            for d in range(EMBED):
                v = plsc.load_gather(tbl_v, [gbase + d])
                plsc.store_scatter(rows_v.at[b],
                                   [bb, jnp.full((16,), d, jnp.int32), s], v)
            return carry
        lax.fori_loop(0, GROUPS, group, 0)

    def process(step, b, refill):
        pltpu.make_async_copy(ids.at[pl.ds(0, CHUNK)], idx_v.at[b],
                              isems[b]).wait()
        @pl.when(step >= NBUF)
        def _():
            pltpu.make_async_copy(rows_v.at[b],
                                  out.at[pl.ds(0, CB)], osems[b]).wait()
        compute_chunk(b)
        pltpu.async_copy(rows_v.at[b],
                         out.at[pl.ds(row_base + step * CB, CB)], osems[b])
        if refill:
            pltpu.async_copy(
                ids.at[pl.ds(id_base + (step + NBUF) * CHUNK, CHUNK)],
                idx_v.at[b], isems[b])

    num_groups = (STEPS - NBUF) // NBUF

    def outer(i, carry):
        for b in range(NBUF):
            process(i * NBUF + b, b, refill=True)
        return carry

    lax.fori_loop(0, num_groups, outer, 0)

    for step in range(num_groups * NBUF, STEPS):
        process(step, step % NBUF, refill=step + NBUF < STEPS)

    for b in range(NBUF):
        pltpu.make_async_copy(rows_v.at[b], out.at[pl.ds(0, CB)],
                              osems[b]).wait()


def kernel(char_ids, embed_weight):
    ids = char_ids.reshape(N).astype(jnp.int32)
    out_bes = _embed_lookup(ids, embed_weight.reshape(VOCAB * EMBED))
    return jnp.transpose(out_bes, (0, 2, 1))
